# baseline (speedup 1.0000x reference)
import functools

import jax
import jax.numpy as jnp
from jax import lax
from jax.experimental import pallas as pl
from jax.experimental.pallas import tpu as pltpu
from jax.experimental.pallas import tpu_sc as plsc

D_EMB = 128
NC, NS = 2, 16
NW = NC * NS
CHUNK = 128
K_PIPE = 5


def _gather_body(table_hbm, idx_hbm, out_hbm, idx_v, rows0, rows1, sem0, sem1):
    wid = lax.axis_index("s") * NC + lax.axis_index("c")
    n_chunks = idx_v.shape[0] // CHUNK
    base = wid * (n_chunks * CHUNK)
    pltpu.sync_copy(idx_hbm.at[wid], idx_v)

    def idx_slice(c):
        return idx_v.at[pl.ds(c * CHUNK, CHUNK)]

    pltpu.async_copy(table_hbm.at[idx_slice(0)], rows0, sem0)
    pltpu.async_copy(table_hbm.at[idx_slice(1)], rows1, sem1)

    def body(i, carry):
        c0 = 2 * i
        pltpu.make_async_copy(table_hbm.at[idx_slice(c0)], rows0, sem0).wait()
        pltpu.sync_copy(rows0, out_hbm.at[pl.ds(base + c0 * CHUNK, CHUNK)])

        @pl.when(c0 + 2 < n_chunks)
        def _():
            pltpu.async_copy(table_hbm.at[idx_slice(c0 + 2)], rows0, sem0)

        pltpu.make_async_copy(table_hbm.at[idx_slice(c0 + 1)], rows1, sem1).wait()
        pltpu.sync_copy(rows1, out_hbm.at[pl.ds(base + (c0 + 1) * CHUNK, CHUNK)])

        @pl.when(c0 + 3 < n_chunks)
        def _():
            pltpu.async_copy(table_hbm.at[idx_slice(c0 + 3)], rows1, sem1)

        return carry

    lax.fori_loop(0, n_chunks // 2, body, 0)

    if n_chunks % 2 == 1:
        c = n_chunks - 1
        pltpu.make_async_copy(table_hbm.at[idx_slice(c)], rows0, sem0).wait()
        pltpu.sync_copy(rows0, out_hbm.at[pl.ds(base + c * CHUNK, CHUNK)])


def _make_gather(n_rows):
    rows_per_w = n_rows // NW
    mesh = plsc.VectorSubcoreMesh(core_axis_name="c", subcore_axis_name="s")
    return pl.kernel(
        _gather_body,
        out_type=jax.ShapeDtypeStruct((n_rows, D_EMB), jnp.float32),
        scratch_types=[
            pltpu.VMEM((rows_per_w,), jnp.int32),
            pltpu.VMEM((CHUNK, D_EMB), jnp.float32),
            pltpu.VMEM((CHUNK, D_EMB), jnp.float32),
            pltpu.SemaphoreType.DMA,
            pltpu.SemaphoreType.DMA,
        ],
        mesh=mesh,
    )


def _mlp_body(lb, B, h_ref, w1_ref, b1_ref, w2_ref, b2_ref, out_ref):
    h = h_ref[...]
    z = lax.dot_general(h, w1_ref[...], (((1,), (1,)), ((), ())),
                        preferred_element_type=jnp.float32)
    z = jnp.maximum(z + b1_ref[...], 0.0)
    o = lax.dot_general(w2_ref[...], z, (((1,), (1,)), ((), ())),
                        preferred_element_type=jnp.float32)
    o = o + b2_ref[...]
    out_ref[...] = o.reshape(o.shape[0], lb, B)


def _mlp_t(h, W1, b1, W2, b2, L, B, l_off, o_prev, lb=8):
    n_tags = W2.shape[0]
    lk = h.shape[0] // B
    grid = (lk // lb,)
    ko = l_off // lb
    args = [h, W1, b1, W2, b2]
    in_specs = [
        pl.BlockSpec((lb * B, D_EMB), lambda i: (i, 0)),
        pl.BlockSpec((D_EMB, D_EMB), lambda i: (0, 0)),
        pl.BlockSpec((1, D_EMB), lambda i: (0, 0)),
        pl.BlockSpec((n_tags, D_EMB), lambda i: (0, 0)),
        pl.BlockSpec((n_tags, 1), lambda i: (0, 0)),
    ]
    aliases = {}
    if o_prev is not None:
        args.append(o_prev)
        in_specs.append(pl.BlockSpec(memory_space=pl.ANY))
        aliases = {5: 0}

    def body(*refs):
        _mlp_body(lb, B, *refs[:5], refs[-1])

    return pl.pallas_call(
        body,
        grid=grid,
        in_specs=in_specs,
        out_specs=pl.BlockSpec((n_tags, lb, B), lambda i: (0, i + ko, 0)),
        out_shape=jax.ShapeDtypeStruct((n_tags, L, B), jnp.float32),
        input_output_aliases=aliases,
    )(*args)


def kernel(x, emb, W1, b1, W2, b2):
    B, L = x.shape
    xT = x.T
    b1r, b2c = b1.reshape(1, -1), b2.reshape(-1, 1)
    lk = L // K_PIPE
    nk = lk * B
    gather = _make_gather(nk)
    o = None
    for k in range(K_PIPE):
        xk = lax.slice_in_dim(xT, k * lk, (k + 1) * lk, axis=0)
        idx = xk.reshape(NW, nk // NW).astype(jnp.int32)
        h = gather(emb, idx)
        o = _mlp_t(h, W1, b1r, W2, b2c, L, B, k * lk, o)
    return jnp.transpose(o, (2, 1, 0))


# speedup vs baseline: 1.0279x; 1.0279x over previous
import functools

import jax
import jax.numpy as jnp
from jax import lax
from jax.experimental import pallas as pl
from jax.experimental.pallas import tpu as pltpu
from jax.experimental.pallas import tpu_sc as plsc

D_EMB = 128
NC, NS = 2, 16
NW = NC * NS
CHUNK = 128
K_PIPE = 5


NBUF = 4


def _gather_body(table_hbm, idx_hbm, out_hbm, idx_v,
                 r0, r1, r2, r3, g0, g1, g2, g3, s0, s1, s2, s3):
    wid = lax.axis_index("s") * NC + lax.axis_index("c")
    n_chunks = idx_v.shape[0] // CHUNK
    base = wid * (n_chunks * CHUNK)
    pltpu.sync_copy(idx_hbm.at[wid], idx_v)

    rows = [r0, r1, r2, r3]
    gs = [g0, g1, g2, g3]
    ss = [s0, s1, s2, s3]

    def idx_slice(c):
        return idx_v.at[pl.ds(c * CHUNK, CHUNK)]

    def out_slice(c):
        return out_hbm.at[pl.ds(base + c * CHUNK, CHUNK)]

    for b in range(NBUF):
        pltpu.async_copy(table_hbm.at[idx_slice(b)], rows[b], gs[b])

    def body(i, carry):
        for b in range(NBUF):
            c = i * NBUF + b
            pltpu.make_async_copy(table_hbm.at[idx_slice(c)], rows[b], gs[b]).wait()
            pltpu.async_copy(rows[b], out_slice(c), ss[b])
            pltpu.make_async_copy(rows[b], out_slice(c), ss[b]).wait()

            @pl.when(c + NBUF < n_chunks)
            def _():
                pltpu.async_copy(table_hbm.at[idx_slice(c + NBUF)], rows[b], gs[b])

        return carry

    lax.fori_loop(0, n_chunks // NBUF, body, 0)

    for b in range(n_chunks % NBUF):
        c = (n_chunks // NBUF) * NBUF + b
        pltpu.make_async_copy(table_hbm.at[idx_slice(c)], rows[b], gs[b]).wait()
        pltpu.sync_copy(rows[b], out_slice(c))


def _make_gather(n_rows):
    rows_per_w = n_rows // NW
    mesh = plsc.VectorSubcoreMesh(core_axis_name="c", subcore_axis_name="s")
    return pl.kernel(
        _gather_body,
        out_type=jax.ShapeDtypeStruct((n_rows, D_EMB), jnp.float32),
        scratch_types=(
            [pltpu.VMEM((rows_per_w,), jnp.int32)]
            + [pltpu.VMEM((CHUNK, D_EMB), jnp.float32)] * NBUF
            + [pltpu.SemaphoreType.DMA] * (2 * NBUF)
        ),
        mesh=mesh,
    )


def _mlp_body(lb, B, h_ref, w1_ref, b1_ref, w2_ref, b2_ref, out_ref):
    h = h_ref[...]
    z = lax.dot_general(h, w1_ref[...], (((1,), (1,)), ((), ())),
                        preferred_element_type=jnp.float32)
    z = jnp.maximum(z + b1_ref[...], 0.0)
    o = lax.dot_general(w2_ref[...], z, (((1,), (1,)), ((), ())),
                        preferred_element_type=jnp.float32)
    o = o + b2_ref[...]
    out_ref[...] = o.reshape(o.shape[0], lb, B)


def _mlp_t(h, W1, b1, W2, b2, L, B, l_off, o_prev, lb=8):
    n_tags = W2.shape[0]
    lk = h.shape[0] // B
    grid = (lk // lb,)
    ko = l_off // lb
    args = [h, W1, b1, W2, b2]
    in_specs = [
        pl.BlockSpec((lb * B, D_EMB), lambda i: (i, 0)),
        pl.BlockSpec((D_EMB, D_EMB), lambda i: (0, 0)),
        pl.BlockSpec((1, D_EMB), lambda i: (0, 0)),
        pl.BlockSpec((n_tags, D_EMB), lambda i: (0, 0)),
        pl.BlockSpec((n_tags, 1), lambda i: (0, 0)),
    ]
    aliases = {}
    if o_prev is not None:
        args.append(o_prev)
        in_specs.append(pl.BlockSpec(memory_space=pl.ANY))
        aliases = {5: 0}

    def body(*refs):
        _mlp_body(lb, B, *refs[:5], refs[-1])

    return pl.pallas_call(
        body,
        grid=grid,
        in_specs=in_specs,
        out_specs=pl.BlockSpec((n_tags, lb, B), lambda i: (0, i + ko, 0)),
        out_shape=jax.ShapeDtypeStruct((n_tags, L, B), jnp.float32),
        input_output_aliases=aliases,
    )(*args)


def kernel(x, emb, W1, b1, W2, b2):
    B, L = x.shape
    xT = x.T
    b1r, b2c = b1.reshape(1, -1), b2.reshape(-1, 1)
    lk = L // K_PIPE
    nk = lk * B
    gather = _make_gather(nk)
    o = None
    for k in range(K_PIPE):
        xk = lax.slice_in_dim(xT, k * lk, (k + 1) * lk, axis=0)
        idx = xk.reshape(NW, nk // NW).astype(jnp.int32)
        h = gather(emb, idx)
        o = _mlp_t(h, W1, b1r, W2, b2c, L, B, k * lk, o)
    return jnp.transpose(o, (2, 1, 0))
